# baseline (speedup 1.0000x reference)
import functools

import jax
import jax.numpy as jnp
from jax import lax
from jax.experimental import pallas as pl
from jax.experimental.pallas import tpu as pltpu
from jax.experimental.pallas import tpu_sc as plsc

NUM_CORES = 2
NUM_SUBCORES = 16
NUM_WORKERS = NUM_CORES * NUM_SUBCORES

BATCH = 16384
HIST_LEN = 20
FEATURES = 64
B = BATCH * HIST_LEN
BLK = 128
NBLK = B // BLK
BLK_PER_W = NBLK // NUM_WORKERS
IDX_PER_W = B // NUM_WORKERS


def _embed_kernel(table_hbm, idx_hbm, out_hbm,
                  idx_v, q_v, rows_v, stage_v, sem0, sem1):
    wid = lax.axis_index("s") * NUM_CORES + lax.axis_index("c")
    sems = (sem0, sem1)

    pltpu.async_copy(
        idx_hbm.at[pl.ds(wid * IDX_PER_W, IDX_PER_W)], idx_v, sem0
    ).wait()

    def start_gather(i, buf):
        for k in range(BLK // 16):
            r = idx_v[pl.ds(i * BLK + k * 16, 16)]
            q_v[buf, pl.ds(k * 16, 16)] = lax.shift_right_logical(r, 1)
        pltpu.async_copy(table_hbm.at[q_v.at[buf]], rows_v.at[buf], sems[buf])

    def wait_gather(buf):
        pltpu.make_async_copy(
            table_hbm.at[q_v.at[buf]], rows_v.at[buf], sems[buf]
        ).wait()

    svec = lax.iota(jnp.int32, 16)
    NG = BLK // 16

    def extract_and_store(i, buf):
        base = i * BLK
        rowv = [svec + 16 * g for g in range(NG)]
        halfv = [
            (idx_v[pl.ds(base + 16 * g, 16)] & 1) * FEATURES
            for g in range(NG)
        ]

        def per_feature(f, _):
            tf = lax.shift_right_logical(f, 3)
            sf = f & 7
            for g in range(NG):
                vals = plsc.load_gather(
                    rows_v.at[buf], [rowv[g], halfv[g] + f]
                )
                stage_v[buf, tf, sf, pl.ds(16 * g, 16)] = vals
            return 0

        lax.fori_loop(0, FEATURES, per_feature, 0, unroll=2)

        blk = wid * BLK_PER_W + i
        h = blk // (BATCH // BLK)
        c = blk % (BATCH // BLK)
        pltpu.sync_copy(
            stage_v.at[buf],
            out_hbm.at[h, :, :, pl.ds(c * BLK, BLK)],
        )

    start_gather(0, 0)

    @pl.loop(0, BLK_PER_W, step=2)
    def _(i):
        start_gather(i + 1, 1)
        wait_gather(0)
        extract_and_store(i, 0)

        @pl.when(i + 2 < BLK_PER_W)
        def _():
            start_gather(i + 2, 0)
        wait_gather(1)
        extract_and_store(i + 1, 1)


@jax.jit
def kernel(inputs, embedding):
    idx_flat = inputs.T.reshape(-1).astype(jnp.int32)
    table2 = embedding.reshape(500000, 128)
    mesh = plsc.VectorSubcoreMesh(
        core_axis_name="c", subcore_axis_name="s",
        num_cores=NUM_CORES, num_subcores=NUM_SUBCORES,
    )
    run = pl.kernel(
        _embed_kernel,
        out_type=jax.ShapeDtypeStruct(
            (HIST_LEN, FEATURES // 8, 8, BATCH), jnp.float32
        ),
        mesh=mesh,
        scratch_types=[
            pltpu.VMEM((IDX_PER_W,), jnp.int32),
            pltpu.VMEM((2, BLK), jnp.int32),
            pltpu.VMEM((2, BLK, 128), jnp.float32),
            pltpu.VMEM((2, FEATURES // 8, 8, BLK), jnp.float32),
            pltpu.SemaphoreType.DMA,
            pltpu.SemaphoreType.DMA,
        ],
        compiler_params=pltpu.CompilerParams(
            use_tc_tiling_on_sc=True, needs_layout_passes=False
        ),
    )
    out = run(table2, idx_flat)
    out = out.reshape(HIST_LEN, FEATURES, BATCH)
    return out.transpose(2, 0, 1)


# speedup vs baseline: 1.3273x; 1.3273x over previous
import functools

import jax
import jax.numpy as jnp
from jax import lax
from jax.experimental import pallas as pl
from jax.experimental.pallas import tpu as pltpu
from jax.experimental.pallas import tpu_sc as plsc

NUM_CORES = 2
NUM_SUBCORES = 16
NUM_WORKERS = NUM_CORES * NUM_SUBCORES

BATCH = 16384
HIST_LEN = 20
FEATURES = 64
B = BATCH * HIST_LEN
B_PER_W = B // NUM_WORKERS
CHUNK = 512
NCHUNK = B_PER_W // CHUNK


def _embed_kernel(table_hbm, idx_hbm, out_hbm, idx_v, rows_v, gsem):
    wid = lax.axis_index("s") * NUM_CORES + lax.axis_index("c")
    base = wid * B_PER_W

    pltpu.sync_copy(idx_hbm.at[pl.ds(base, B_PER_W)], idx_v)

    def start_gather(i, buf):
        pltpu.async_copy(
            table_hbm.at[idx_v.at[pl.ds(i * CHUNK, CHUNK)]],
            rows_v.at[buf],
            gsem,
        )

    def finish_and_store(i, buf):
        pltpu.make_async_copy(
            table_hbm.at[idx_v.at[pl.ds(i * CHUNK, CHUNK)]],
            rows_v.at[buf],
            gsem,
        ).wait()
        pltpu.sync_copy(rows_v.at[buf], out_hbm.at[pl.ds(base + i * CHUNK, CHUNK)])

    start_gather(0, 0)

    @pl.loop(0, NCHUNK, step=2)
    def _(i):
        start_gather(i + 1, 1)
        finish_and_store(i, 0)
        @pl.when(i + 2 < NCHUNK)
        def _():
            start_gather(i + 2, 0)
        finish_and_store(i + 1, 1)


@jax.jit
def kernel(inputs, embedding):
    idx_flat = inputs.T.reshape(-1).astype(jnp.int32)
    mesh = plsc.VectorSubcoreMesh(
        core_axis_name="c", subcore_axis_name="s",
        num_cores=NUM_CORES, num_subcores=NUM_SUBCORES,
    )
    run = pl.kernel(
        _embed_kernel,
        out_type=jax.ShapeDtypeStruct((B, FEATURES), jnp.float32),
        mesh=mesh,
        scratch_types=[
            pltpu.VMEM((B_PER_W,), jnp.int32),
            pltpu.VMEM((2, CHUNK, FEATURES), jnp.float32),
            pltpu.SemaphoreType.DMA,
        ],
        compiler_params=pltpu.CompilerParams(use_tc_tiling_on_sc=False),
    )
    out = run(embedding, idx_flat)
    return out.reshape(HIST_LEN, BATCH, FEATURES).transpose(1, 0, 2)


# speedup vs baseline: 1.3297x; 1.0018x over previous
import functools

import jax
import jax.numpy as jnp
from jax import lax
from jax.experimental import pallas as pl
from jax.experimental.pallas import tpu as pltpu
from jax.experimental.pallas import tpu_sc as plsc

NUM_CORES = 2
NUM_SUBCORES = 16
NUM_WORKERS = NUM_CORES * NUM_SUBCORES

BATCH = 16384
HIST_LEN = 20
FEATURES = 64
B = BATCH * HIST_LEN
B_PER_W = B // NUM_WORKERS
CHUNK = 640
NCHUNK = B_PER_W // CHUNK


def _embed_kernel(table_hbm, idx_hbm, out_hbm, idx_v, rows_v, gsem):
    wid = lax.axis_index("s") * NUM_CORES + lax.axis_index("c")
    base = wid * B_PER_W

    pltpu.sync_copy(idx_hbm.at[pl.ds(base, B_PER_W)], idx_v)

    def start_gather(i, buf):
        pltpu.async_copy(
            table_hbm.at[idx_v.at[pl.ds(i * CHUNK, CHUNK)]],
            rows_v.at[buf],
            gsem,
        )

    def finish_and_store(i, buf):
        pltpu.make_async_copy(
            table_hbm.at[idx_v.at[pl.ds(i * CHUNK, CHUNK)]],
            rows_v.at[buf],
            gsem,
        ).wait()
        pltpu.sync_copy(rows_v.at[buf], out_hbm.at[pl.ds(base + i * CHUNK, CHUNK)])

    start_gather(0, 0)

    @pl.loop(0, NCHUNK, step=2)
    def _(i):
        start_gather(i + 1, 1)
        finish_and_store(i, 0)
        @pl.when(i + 2 < NCHUNK)
        def _():
            start_gather(i + 2, 0)
        finish_and_store(i + 1, 1)


@jax.jit
def kernel(inputs, embedding):
    idx_flat = inputs.T.reshape(-1).astype(jnp.int32)
    mesh = plsc.VectorSubcoreMesh(
        core_axis_name="c", subcore_axis_name="s",
        num_cores=NUM_CORES, num_subcores=NUM_SUBCORES,
    )
    run = pl.kernel(
        _embed_kernel,
        out_type=jax.ShapeDtypeStruct((B, FEATURES), jnp.float32),
        mesh=mesh,
        scratch_types=[
            pltpu.VMEM((B_PER_W,), jnp.int32),
            pltpu.VMEM((2, CHUNK, FEATURES), jnp.float32),
            pltpu.SemaphoreType.DMA,
        ],
        compiler_params=pltpu.CompilerParams(use_tc_tiling_on_sc=False),
    )
    out = run(embedding, idx_flat)
    return out.reshape(HIST_LEN, BATCH, FEATURES).transpose(1, 0, 2)
